# Initial kernel scaffold; baseline (speedup 1.0000x reference)
import jax
import jax.numpy as jnp
from jax.experimental import pallas as pl


def kernel(x, edge_index, batch, W_l1, b1, W_r1, g1, be1, W_l2, b2, W_r2, g2, be2, Wc1, bc1, gc, bec, Wc2, bc2):
    raise NotImplementedError("write your pallas kernel here")


# speedup vs baseline: 2.9638x; 2.9638x over previous
import functools

import jax
import jax.numpy as jnp
from jax import lax
from jax.experimental import pallas as pl
from jax.experimental.pallas import tpu as pltpu
from jax.experimental.pallas import tpu_sc as plsc

N = 10000
E = 320000
DIN = 128
DH = 256
B = 100

NP = 10240
EP = 327680
NC = 2
NS = 16
CHUNK = 128
CPT = EP // NS // CHUNK
RPT = NP // NS
FW = 64
TCB = 1024
NTCB = NP // TCB


def _sage1_body(xa_ref, xb_ref, src_ref, dst_ref, z64_ref, z8_ref, ones_ref,
                suma_ref, sumb_ref, deg_ref, srcb, dstb, rows, ones, acc,
                dacc, sem):
  c = lax.axis_index("c")
  s = lax.axis_index("s")
  rsl = pl.ds(s * RPT, RPT)
  pltpu.sync_copy(z64_ref.at[rsl], acc.at[rsl])
  pltpu.sync_copy(z8_ref.at[rsl], dacc.at[rsl])
  pltpu.sync_copy(src_ref.at[pl.ds(s * CPT, CPT)], srcb)
  pltpu.sync_copy(dst_ref.at[pl.ds(s * CPT, CPT)], dstb)
  pltpu.sync_copy(ones_ref, ones)
  plsc.subcore_barrier()

  def step(j, carry):
    @pl.when(c == 0)
    def _():
      pltpu.async_copy(xa_ref.at[srcb.at[j]], rows, sem).wait()
      pltpu.sync_copy(ones, dacc.at[dstb.at[j]], add=True)

    @pl.when(c == 1)
    def _():
      pltpu.async_copy(xb_ref.at[srcb.at[j]], rows, sem).wait()

    pltpu.sync_copy(rows, acc.at[dstb.at[j]], add=True)
    return carry

  lax.fori_loop(0, CPT, step, 0)
  plsc.subcore_barrier()

  @pl.when(c == 0)
  def _():
    pltpu.sync_copy(acc.at[rsl], suma_ref.at[rsl])
    pltpu.sync_copy(dacc.at[rsl], deg_ref.at[rsl])

  @pl.when(c == 1)
  def _():
    pltpu.sync_copy(acc.at[rsl], sumb_ref.at[rsl])


def _sage2_body(h0_ref, h1_ref, h2_ref, h3_ref, src_ref, dst_ref, z64_ref,
                s2a_ref, s2b_ref, s2c_ref, s2d_ref, srcb, dstb, rows, acc,
                sem):
  c = lax.axis_index("c")
  s = lax.axis_index("s")
  rsl = pl.ds(s * RPT, RPT)
  pltpu.sync_copy(src_ref.at[pl.ds(s * CPT, CPT)], srcb)
  pltpu.sync_copy(dst_ref.at[pl.ds(s * CPT, CPT)], dstb)

  outs = (s2a_ref, s2b_ref, s2c_ref, s2d_ref)
  for p in range(2):
    tab_c0 = h0_ref if p == 0 else h1_ref
    tab_c1 = h2_ref if p == 0 else h3_ref
    pltpu.sync_copy(z64_ref.at[rsl], acc.at[rsl])
    plsc.subcore_barrier()

    def step(j, carry):
      @pl.when(c == 0)
      def _():
        pltpu.async_copy(tab_c0.at[srcb.at[j]], rows, sem).wait()

      @pl.when(c == 1)
      def _():
        pltpu.async_copy(tab_c1.at[srcb.at[j]], rows, sem).wait()

      pltpu.sync_copy(rows, acc.at[dstb.at[j]], add=True)
      return carry

    lax.fori_loop(0, CPT, step, 0)
    plsc.subcore_barrier()

    @pl.when(c == 0)
    def _():
      pltpu.sync_copy(acc.at[rsl], outs[p].at[rsl])

    @pl.when(c == 1)
    def _():
      pltpu.sync_copy(acc.at[rsl], outs[2 + p].at[rsl])


@functools.cache
def _get_sc_kernels():
  mesh = plsc.VectorSubcoreMesh(
      core_axis_name="c", subcore_axis_name="s",
      num_cores=NC, num_subcores=NS)
  sage1 = pl.kernel(
      _sage1_body,
      out_type=[
          jax.ShapeDtypeStruct((NP, FW), jnp.float32),
          jax.ShapeDtypeStruct((NP, FW), jnp.float32),
          jax.ShapeDtypeStruct((NP, 8), jnp.float32),
      ],
      mesh=mesh,
      compiler_params=pltpu.CompilerParams(use_tc_tiling_on_sc=False),
      scratch_types=[
          pltpu.VMEM((CPT, CHUNK), jnp.int32),
          pltpu.VMEM((CPT, CHUNK), jnp.int32),
          pltpu.VMEM((CHUNK, FW), jnp.float32),
          pltpu.VMEM((CHUNK, 8), jnp.float32),
          pltpu.VMEM_SHARED((NP, FW), jnp.float32),
          pltpu.VMEM_SHARED((NP, 8), jnp.float32),
          pltpu.SemaphoreType.DMA,
      ],
  )
  sage2 = pl.kernel(
      _sage2_body,
      out_type=[jax.ShapeDtypeStruct((NP, FW), jnp.float32)] * 4,
      mesh=mesh,
      compiler_params=pltpu.CompilerParams(use_tc_tiling_on_sc=False),
      scratch_types=[
          pltpu.VMEM((CPT, CHUNK), jnp.int32),
          pltpu.VMEM((CPT, CHUNK), jnp.int32),
          pltpu.VMEM((CHUNK, FW), jnp.float32),
          pltpu.VMEM_SHARED((NP, FW), jnp.float32),
          pltpu.SemaphoreType.DMA,
      ],
  )
  return sage1, sage2


def _dbf(a, b):
  return jnp.dot(a.astype(jnp.bfloat16), b.astype(jnp.bfloat16),
                 preferred_element_type=jnp.float32)


def _row_mask(i, z):
  rid = i * TCB + lax.broadcasted_iota(jnp.int32, (TCB, 1), 0)
  return jnp.where(rid < N, z, 0.0)


def _acc_stats(i, st_ref, z):
  zm = _row_mask(i, z)
  upd = jnp.concatenate(
      [jnp.sum(zm, axis=0)[None, :], jnp.sum(zm * zm, axis=0)[None, :]], axis=0)

  @pl.when(i == 0)
  def _():
    st_ref[...] = upd

  @pl.when(i > 0)
  def _():
    st_ref[...] = st_ref[...] + upd


def _z1_body(sa_ref, sb_ref, d_ref, x_ref, wl_ref, wr_ref, b_ref,
             z_ref, st_ref):
  i = pl.program_id(0)
  rdeg = 1.0 / jnp.maximum(d_ref[:, 0:1], 1.0)
  z = (_dbf(sa_ref[...] * rdeg, wl_ref[0:FW, :])
       + _dbf(sb_ref[...] * rdeg, wl_ref[FW:, :])
       + _dbf(x_ref[...], wr_ref[...])
       + b_ref[...])
  z_ref[...] = z
  _acc_stats(i, st_ref, z)


def _bn_split_body(z_ref, st_ref, g_ref, be_ref,
                   h0_ref, h1_ref, h2_ref, h3_ref):
  m = st_ref[0:1, :] / float(N)
  var = st_ref[1:2, :] / float(N) - m * m
  rstd = (1.0 / jnp.sqrt(var + 1e-5))
  h = jnp.maximum((z_ref[...] - m) * rstd * g_ref[...] + be_ref[...], 0.0)
  h0_ref[...] = h[:, 0:FW]
  h1_ref[...] = h[:, FW:2 * FW]
  h2_ref[...] = h[:, 2 * FW:3 * FW]
  h3_ref[...] = h[:, 3 * FW:]


def _z2_body(sa_ref, sb_ref, sc_ref, sd_ref, d_ref,
             h0_ref, h1_ref, h2_ref, h3_ref,
             wl_ref, wr_ref, b_ref, z_ref, st_ref):
  i = pl.program_id(0)
  rdeg = 1.0 / jnp.maximum(d_ref[:, 0:1], 1.0)
  z = b_ref[...] + jnp.zeros((TCB, DH), jnp.float32)
  for g, (s_ref, h_ref) in enumerate(
      zip((sa_ref, sb_ref, sc_ref, sd_ref), (h0_ref, h1_ref, h2_ref, h3_ref))):
    wsl = slice(g * FW, (g + 1) * FW)
    z = z + _dbf(s_ref[...] * rdeg, wl_ref[wsl, :])
    z = z + _dbf(h_ref[...], wr_ref[wsl, :])
  z_ref[...] = z
  _acc_stats(i, st_ref, z)


def _bn_full_body(z_ref, st_ref, g_ref, be_ref, h_ref):
  m = st_ref[0:1, :] / float(N)
  var = st_ref[1:2, :] / float(N) - m * m
  rstd = (1.0 / jnp.sqrt(var + 1e-5))
  h_ref[...] = jnp.maximum(
      (z_ref[...] - m) * rstd * g_ref[...] + be_ref[...], 0.0)


def _head_body(h_ref, bat_ref, wc1_ref, bc1_ref, gc_ref, bec_ref,
               wc2_ref, bc2_ref, out_ref, pacc, cacc):
  i = pl.program_id(0)
  cols = lax.broadcasted_iota(jnp.int32, (TCB, 128), 1)
  p = (bat_ref[...] == cols).astype(jnp.float32)
  contrib = lax.dot_general(p, h_ref[...], (((0,), (0,)), ((), ())),
                            preferred_element_type=jnp.float32, precision=lax.Precision.HIGHEST)
  cnt = jnp.sum(p, axis=0)[:, None]

  @pl.when(i == 0)
  def _():
    pacc[...] = contrib
    cacc[...] = cnt

  @pl.when(i > 0)
  def _():
    pacc[...] = pacc[...] + contrib
    cacc[...] = cacc[...] + cnt

  @pl.when(i == NTCB - 1)
  def _():
    pooled = pacc[...] / jnp.maximum(cacc[...], 1.0)
    c1 = jnp.maximum(
        _dbf(pooled, wc1_ref[...])
        + bc1_ref[...], 0.0)
    rmask = lax.broadcasted_iota(jnp.int32, (128, 1), 0) < B
    c1 = jnp.where(rmask, c1, 0.0)
    m = jnp.sum(c1, axis=0, keepdims=True) / float(B)
    var = jnp.sum(c1 * c1, axis=0, keepdims=True) / float(B) - m * m
    cn = (c1 - m) * (1.0 / jnp.sqrt(var + 1e-5)) * gc_ref[...] + bec_ref[...]
    out_ref[...] = _dbf(cn, wc2_ref[...]) + bc2_ref[...]


def _rows(i):
  return (i, 0)


def _const(i):
  return (0, 0)


def kernel(x, edge_index, batch, W_l1, b1, W_r1, g1, be1, W_l2, b2, W_r2,
           g2, be2, Wc1, bc1, gc, bec, Wc2, bc2):
  f32 = jnp.float32
  src = jnp.concatenate(
      [edge_index[0].astype(jnp.int32), jnp.zeros((EP - E,), jnp.int32)])
  dst = jnp.concatenate(
      [edge_index[1].astype(jnp.int32),
       jnp.full((EP - E,), NP - 1, jnp.int32)])
  src2 = src.reshape(-1, CHUNK)
  dst2 = dst.reshape(-1, CHUNK)
  xp = jnp.concatenate([x, jnp.zeros((NP - N, DIN), f32)], axis=0)
  xa = xp[:, :FW]
  xb = xp[:, FW:]
  batp = jnp.concatenate(
      [batch.astype(jnp.int32), jnp.full((NP - N,), 1000, jnp.int32)]
  ).reshape(NP, 1)
  z64 = jnp.zeros((NP, FW), f32)
  z8 = jnp.zeros((NP, 8), f32)
  ones8 = jnp.ones((CHUNK, 8), f32)
  b1r = b1.reshape(1, DH)
  b2r = b2.reshape(1, DH)
  g1r, be1r = g1.reshape(1, DH), be1.reshape(1, DH)
  g2r, be2r = g2.reshape(1, DH), be2.reshape(1, DH)
  bc1r = bc1.reshape(1, DH)
  gcr, becr = gc.reshape(1, DH), bec.reshape(1, DH)
  wc2p = jnp.concatenate([Wc2, jnp.zeros((DH, 128 - Wc2.shape[1]), f32)], 1)
  bc2p = jnp.concatenate([bc2, jnp.zeros((128 - bc2.shape[0],), f32)]
                         ).reshape(1, 128)

  sage1, sage2 = _get_sc_kernels()
  s1a, s1b, deg = sage1(xa, xb, src2, dst2, z64, z8, ones8)

  hblk = pl.BlockSpec((TCB, DIN), _rows)
  gblk = pl.BlockSpec((TCB, FW), _rows)
  dblk = pl.BlockSpec((TCB, 8), _rows)
  zblk = pl.BlockSpec((TCB, DH), _rows)
  stblk = pl.BlockSpec((2, DH), _const)
  w128 = pl.BlockSpec((DIN, DH), _const)
  w256 = pl.BlockSpec((DH, DH), _const)
  vec = pl.BlockSpec((1, DH), _const)

  z1, st1 = pl.pallas_call(
      _z1_body,
      grid=(NTCB,),
      in_specs=[gblk, gblk, dblk, hblk, w128, w128, vec],
      out_specs=[zblk, stblk],
      out_shape=[jax.ShapeDtypeStruct((NP, DH), f32),
                 jax.ShapeDtypeStruct((2, DH), f32)],
  )(s1a, s1b, deg, xp, W_l1, W_r1, b1r)

  h1s = pl.pallas_call(
      _bn_split_body,
      grid=(NTCB,),
      in_specs=[zblk, stblk, vec, vec],
      out_specs=[gblk, gblk, gblk, gblk],
      out_shape=[jax.ShapeDtypeStruct((NP, FW), f32)] * 4,
  )(z1, st1, g1r, be1r)

  s2 = sage2(h1s[0], h1s[1], h1s[2], h1s[3], src2, dst2, z64)

  z2, st2 = pl.pallas_call(
      _z2_body,
      grid=(NTCB,),
      in_specs=[gblk, gblk, gblk, gblk, dblk,
                gblk, gblk, gblk, gblk, w256, w256, vec],
      out_specs=[zblk, stblk],
      out_shape=[jax.ShapeDtypeStruct((NP, DH), f32),
                 jax.ShapeDtypeStruct((2, DH), f32)],
  )(s2[0], s2[1], s2[2], s2[3], deg,
    h1s[0], h1s[1], h1s[2], h1s[3], W_l2, W_r2, b2r)

  h2 = pl.pallas_call(
      _bn_full_body,
      grid=(NTCB,),
      in_specs=[zblk, stblk, vec, vec],
      out_specs=zblk,
      out_shape=jax.ShapeDtypeStruct((NP, DH), f32),
  )(z2, st2, g2r, be2r)

  outp = pl.pallas_call(
      _head_body,
      grid=(NTCB,),
      in_specs=[zblk, pl.BlockSpec((TCB, 1), _rows), w256, vec, vec, vec,
                pl.BlockSpec((DH, 128), _const), pl.BlockSpec((1, 128), _const)],
      out_specs=pl.BlockSpec((128, 128), _const),
      out_shape=jax.ShapeDtypeStruct((128, 128), f32),
      scratch_shapes=[pltpu.VMEM((128, DH), f32), pltpu.VMEM((128, 1), f32)],
  )(h2, batp, Wc1, bc1r, gcr, becr, wc2p, bc2p)

  return outp[:B, :2]


# speedup vs baseline: 3.6662x; 1.2370x over previous
import functools

import jax
import jax.numpy as jnp
from jax import lax
from jax.experimental import pallas as pl
from jax.experimental.pallas import tpu as pltpu
from jax.experimental.pallas import tpu_sc as plsc

N = 10000
E = 320000
DIN = 128
DH = 256
B = 100

NP = 10240
EP = 327680
NC = 2
NS = 16
CHUNK = 128
CPT = EP // NS // CHUNK
RPT = NP // NS
FW = 64
TCB = 1024
NTCB = NP // TCB


KB = 2
NG = CPT // KB


def _sweep(c, tab0, tab1, srcb, dstb, rowsA, rowsB, semA, semB, acc,
           deg_pair=None):
  def fire(bank, sem, g):
    for b in range(KB):
      ch = g * KB + b

      @pl.when(c == 0)
      def _():
        pltpu.async_copy(tab0.at[srcb.at[ch]], bank.at[b], sem)

      @pl.when(c == 1)
      def _():
        pltpu.async_copy(tab1.at[srcb.at[ch]], bank.at[b], sem)

  def drain(bank, sem):
    for b in range(KB):
      pltpu.make_async_copy(tab0.at[srcb.at[0]], bank.at[b], sem).wait()

  def scatter(bank, g):
    for b in range(KB):
      ch = g * KB + b
      pltpu.sync_copy(bank.at[b], acc.at[dstb.at[ch]], add=True)
      if deg_pair is not None:
        ones, dacc = deg_pair

        @pl.when(c == 0)
        def _():
          pltpu.sync_copy(ones, dacc.at[dstb.at[ch]], add=True)

  fire(rowsA, semA, 0)

  def body(j2, carry):
    g0 = 2 * j2
    drain(rowsA, semA)
    fire(rowsB, semB, g0 + 1)
    scatter(rowsA, g0)
    drain(rowsB, semB)

    @pl.when(j2 + 1 < NG // 2)
    def _():
      fire(rowsA, semA, g0 + 2)

    scatter(rowsB, g0 + 1)
    return carry

  lax.fori_loop(0, NG // 2, body, 0)


def _sage1_body(xa_ref, xb_ref, src_ref, dst_ref, z64_ref, z8_ref, ones_ref,
                suma_ref, sumb_ref, deg_ref, srcb, dstb, rowsA, rowsB, ones,
                acc, dacc, semA, semB):
  c = lax.axis_index("c")
  s = lax.axis_index("s")
  rsl = pl.ds(s * RPT, RPT)
  pltpu.sync_copy(z64_ref.at[rsl], acc.at[rsl])
  pltpu.sync_copy(z8_ref.at[rsl], dacc.at[rsl])
  pltpu.sync_copy(src_ref.at[pl.ds(s * CPT, CPT)], srcb)
  pltpu.sync_copy(dst_ref.at[pl.ds(s * CPT, CPT)], dstb)
  pltpu.sync_copy(ones_ref, ones)
  plsc.subcore_barrier()

  _sweep(c, xa_ref, xb_ref, srcb, dstb, rowsA, rowsB, semA, semB, acc,
         deg_pair=(ones, dacc))
  plsc.subcore_barrier()

  @pl.when(c == 0)
  def _():
    pltpu.sync_copy(acc.at[rsl], suma_ref.at[rsl])
    pltpu.sync_copy(dacc.at[rsl], deg_ref.at[rsl])

  @pl.when(c == 1)
  def _():
    pltpu.sync_copy(acc.at[rsl], sumb_ref.at[rsl])


def _sage2_body(h0_ref, h1_ref, h2_ref, h3_ref, src_ref, dst_ref, z64_ref,
                s2a_ref, s2b_ref, s2c_ref, s2d_ref, srcb, dstb, rowsA, rowsB,
                acc, semA, semB):
  c = lax.axis_index("c")
  s = lax.axis_index("s")
  rsl = pl.ds(s * RPT, RPT)
  pltpu.sync_copy(src_ref.at[pl.ds(s * CPT, CPT)], srcb)
  pltpu.sync_copy(dst_ref.at[pl.ds(s * CPT, CPT)], dstb)

  outs = (s2a_ref, s2b_ref, s2c_ref, s2d_ref)
  for p in range(2):
    tab_c0 = h0_ref if p == 0 else h1_ref
    tab_c1 = h2_ref if p == 0 else h3_ref
    pltpu.sync_copy(z64_ref.at[rsl], acc.at[rsl])
    plsc.subcore_barrier()
    _sweep(c, tab_c0, tab_c1, srcb, dstb, rowsA, rowsB, semA, semB, acc)
    plsc.subcore_barrier()

    @pl.when(c == 0)
    def _():
      pltpu.sync_copy(acc.at[rsl], outs[p].at[rsl])

    @pl.when(c == 1)
    def _():
      pltpu.sync_copy(acc.at[rsl], outs[2 + p].at[rsl])


@functools.cache
def _get_sc_kernels():
  mesh = plsc.VectorSubcoreMesh(
      core_axis_name="c", subcore_axis_name="s",
      num_cores=NC, num_subcores=NS)
  sage1 = pl.kernel(
      _sage1_body,
      out_type=[
          jax.ShapeDtypeStruct((NP, FW), jnp.float32),
          jax.ShapeDtypeStruct((NP, FW), jnp.float32),
          jax.ShapeDtypeStruct((NP, 8), jnp.float32),
      ],
      mesh=mesh,
      compiler_params=pltpu.CompilerParams(use_tc_tiling_on_sc=False),
      scratch_types=[
          pltpu.VMEM((CPT, CHUNK), jnp.int32),
          pltpu.VMEM((CPT, CHUNK), jnp.int32),
          pltpu.VMEM((KB, CHUNK, FW), jnp.float32),
          pltpu.VMEM((KB, CHUNK, FW), jnp.float32),
          pltpu.VMEM((CHUNK, 8), jnp.float32),
          pltpu.VMEM_SHARED((NP, FW), jnp.float32),
          pltpu.VMEM_SHARED((NP, 8), jnp.float32),
          pltpu.SemaphoreType.DMA,
          pltpu.SemaphoreType.DMA,
      ],
  )
  sage2 = pl.kernel(
      _sage2_body,
      out_type=[jax.ShapeDtypeStruct((NP, FW), jnp.float32)] * 4,
      mesh=mesh,
      compiler_params=pltpu.CompilerParams(use_tc_tiling_on_sc=False),
      scratch_types=[
          pltpu.VMEM((CPT, CHUNK), jnp.int32),
          pltpu.VMEM((CPT, CHUNK), jnp.int32),
          pltpu.VMEM((KB, CHUNK, FW), jnp.float32),
          pltpu.VMEM((KB, CHUNK, FW), jnp.float32),
          pltpu.VMEM_SHARED((NP, FW), jnp.float32),
          pltpu.SemaphoreType.DMA,
          pltpu.SemaphoreType.DMA,
      ],
  )
  return sage1, sage2


def _dbf(a, b):
  return jnp.dot(a.astype(jnp.bfloat16), b.astype(jnp.bfloat16),
                 preferred_element_type=jnp.float32)


def _row_mask(i, z):
  rid = i * TCB + lax.broadcasted_iota(jnp.int32, (TCB, 1), 0)
  return jnp.where(rid < N, z, 0.0)


def _acc_stats(i, st_ref, z):
  zm = _row_mask(i, z)
  upd = jnp.concatenate(
      [jnp.sum(zm, axis=0)[None, :], jnp.sum(zm * zm, axis=0)[None, :]], axis=0)

  @pl.when(i == 0)
  def _():
    st_ref[...] = upd

  @pl.when(i > 0)
  def _():
    st_ref[...] = st_ref[...] + upd


def _z1_body(sa_ref, sb_ref, d_ref, x_ref, wl_ref, wr_ref, b_ref,
             z_ref, st_ref):
  i = pl.program_id(0)
  rdeg = 1.0 / jnp.maximum(d_ref[:, 0:1], 1.0)
  z = (_dbf(sa_ref[...] * rdeg, wl_ref[0:FW, :])
       + _dbf(sb_ref[...] * rdeg, wl_ref[FW:, :])
       + _dbf(x_ref[...], wr_ref[...])
       + b_ref[...])
  z_ref[...] = z
  _acc_stats(i, st_ref, z)


def _bn_split_body(z_ref, st_ref, g_ref, be_ref,
                   h0_ref, h1_ref, h2_ref, h3_ref):
  m = st_ref[0:1, :] / float(N)
  var = st_ref[1:2, :] / float(N) - m * m
  rstd = (1.0 / jnp.sqrt(var + 1e-5))
  h = jnp.maximum((z_ref[...] - m) * rstd * g_ref[...] + be_ref[...], 0.0)
  h0_ref[...] = h[:, 0:FW]
  h1_ref[...] = h[:, FW:2 * FW]
  h2_ref[...] = h[:, 2 * FW:3 * FW]
  h3_ref[...] = h[:, 3 * FW:]


def _z2_body(sa_ref, sb_ref, sc_ref, sd_ref, d_ref,
             h0_ref, h1_ref, h2_ref, h3_ref,
             wl_ref, wr_ref, b_ref, z_ref, st_ref):
  i = pl.program_id(0)
  rdeg = 1.0 / jnp.maximum(d_ref[:, 0:1], 1.0)
  z = b_ref[...] + jnp.zeros((TCB, DH), jnp.float32)
  for g, (s_ref, h_ref) in enumerate(
      zip((sa_ref, sb_ref, sc_ref, sd_ref), (h0_ref, h1_ref, h2_ref, h3_ref))):
    wsl = slice(g * FW, (g + 1) * FW)
    z = z + _dbf(s_ref[...] * rdeg, wl_ref[wsl, :])
    z = z + _dbf(h_ref[...], wr_ref[wsl, :])
  z_ref[...] = z
  _acc_stats(i, st_ref, z)


def _bn_full_body(z_ref, st_ref, g_ref, be_ref, h_ref):
  m = st_ref[0:1, :] / float(N)
  var = st_ref[1:2, :] / float(N) - m * m
  rstd = (1.0 / jnp.sqrt(var + 1e-5))
  h_ref[...] = jnp.maximum(
      (z_ref[...] - m) * rstd * g_ref[...] + be_ref[...], 0.0)


def _head_body(h_ref, bat_ref, wc1_ref, bc1_ref, gc_ref, bec_ref,
               wc2_ref, bc2_ref, out_ref, pacc, cacc):
  i = pl.program_id(0)
  cols = lax.broadcasted_iota(jnp.int32, (TCB, 128), 1)
  p = (bat_ref[...] == cols).astype(jnp.float32)
  contrib = lax.dot_general(p, h_ref[...], (((0,), (0,)), ((), ())),
                            preferred_element_type=jnp.float32, precision=lax.Precision.HIGHEST)
  cnt = jnp.sum(p, axis=0)[:, None]

  @pl.when(i == 0)
  def _():
    pacc[...] = contrib
    cacc[...] = cnt

  @pl.when(i > 0)
  def _():
    pacc[...] = pacc[...] + contrib
    cacc[...] = cacc[...] + cnt

  @pl.when(i == NTCB - 1)
  def _():
    pooled = pacc[...] / jnp.maximum(cacc[...], 1.0)
    c1 = jnp.maximum(
        _dbf(pooled, wc1_ref[...])
        + bc1_ref[...], 0.0)
    rmask = lax.broadcasted_iota(jnp.int32, (128, 1), 0) < B
    c1 = jnp.where(rmask, c1, 0.0)
    m = jnp.sum(c1, axis=0, keepdims=True) / float(B)
    var = jnp.sum(c1 * c1, axis=0, keepdims=True) / float(B) - m * m
    cn = (c1 - m) * (1.0 / jnp.sqrt(var + 1e-5)) * gc_ref[...] + bec_ref[...]
    out_ref[...] = _dbf(cn, wc2_ref[...]) + bc2_ref[...]


def _rows(i):
  return (i, 0)


def _const(i):
  return (0, 0)


def kernel(x, edge_index, batch, W_l1, b1, W_r1, g1, be1, W_l2, b2, W_r2,
           g2, be2, Wc1, bc1, gc, bec, Wc2, bc2):
  f32 = jnp.float32
  src = jnp.concatenate(
      [edge_index[0].astype(jnp.int32), jnp.zeros((EP - E,), jnp.int32)])
  dst = jnp.concatenate(
      [edge_index[1].astype(jnp.int32),
       jnp.full((EP - E,), NP - 1, jnp.int32)])
  src2 = src.reshape(-1, CHUNK)
  dst2 = dst.reshape(-1, CHUNK)
  xp = jnp.concatenate([x, jnp.zeros((NP - N, DIN), f32)], axis=0)
  xa = xp[:, :FW]
  xb = xp[:, FW:]
  batp = jnp.concatenate(
      [batch.astype(jnp.int32), jnp.full((NP - N,), 1000, jnp.int32)]
  ).reshape(NP, 1)
  z64 = jnp.zeros((NP, FW), f32)
  z8 = jnp.zeros((NP, 8), f32)
  ones8 = jnp.ones((CHUNK, 8), f32)
  b1r = b1.reshape(1, DH)
  b2r = b2.reshape(1, DH)
  g1r, be1r = g1.reshape(1, DH), be1.reshape(1, DH)
  g2r, be2r = g2.reshape(1, DH), be2.reshape(1, DH)
  bc1r = bc1.reshape(1, DH)
  gcr, becr = gc.reshape(1, DH), bec.reshape(1, DH)
  wc2p = jnp.concatenate([Wc2, jnp.zeros((DH, 128 - Wc2.shape[1]), f32)], 1)
  bc2p = jnp.concatenate([bc2, jnp.zeros((128 - bc2.shape[0],), f32)]
                         ).reshape(1, 128)

  sage1, sage2 = _get_sc_kernels()
  s1a, s1b, deg = sage1(xa, xb, src2, dst2, z64, z8, ones8)

  hblk = pl.BlockSpec((TCB, DIN), _rows)
  gblk = pl.BlockSpec((TCB, FW), _rows)
  dblk = pl.BlockSpec((TCB, 8), _rows)
  zblk = pl.BlockSpec((TCB, DH), _rows)
  stblk = pl.BlockSpec((2, DH), _const)
  w128 = pl.BlockSpec((DIN, DH), _const)
  w256 = pl.BlockSpec((DH, DH), _const)
  vec = pl.BlockSpec((1, DH), _const)

  z1, st1 = pl.pallas_call(
      _z1_body,
      grid=(NTCB,),
      in_specs=[gblk, gblk, dblk, hblk, w128, w128, vec],
      out_specs=[zblk, stblk],
      out_shape=[jax.ShapeDtypeStruct((NP, DH), f32),
                 jax.ShapeDtypeStruct((2, DH), f32)],
  )(s1a, s1b, deg, xp, W_l1, W_r1, b1r)

  h1s = pl.pallas_call(
      _bn_split_body,
      grid=(NTCB,),
      in_specs=[zblk, stblk, vec, vec],
      out_specs=[gblk, gblk, gblk, gblk],
      out_shape=[jax.ShapeDtypeStruct((NP, FW), f32)] * 4,
  )(z1, st1, g1r, be1r)

  s2 = sage2(h1s[0], h1s[1], h1s[2], h1s[3], src2, dst2, z64)

  z2, st2 = pl.pallas_call(
      _z2_body,
      grid=(NTCB,),
      in_specs=[gblk, gblk, gblk, gblk, dblk,
                gblk, gblk, gblk, gblk, w256, w256, vec],
      out_specs=[zblk, stblk],
      out_shape=[jax.ShapeDtypeStruct((NP, DH), f32),
                 jax.ShapeDtypeStruct((2, DH), f32)],
  )(s2[0], s2[1], s2[2], s2[3], deg,
    h1s[0], h1s[1], h1s[2], h1s[3], W_l2, W_r2, b2r)

  h2 = pl.pallas_call(
      _bn_full_body,
      grid=(NTCB,),
      in_specs=[zblk, stblk, vec, vec],
      out_specs=zblk,
      out_shape=jax.ShapeDtypeStruct((NP, DH), f32),
  )(z2, st2, g2r, be2r)

  outp = pl.pallas_call(
      _head_body,
      grid=(NTCB,),
      in_specs=[zblk, pl.BlockSpec((TCB, 1), _rows), w256, vec, vec, vec,
                pl.BlockSpec((DH, 128), _const), pl.BlockSpec((1, 128), _const)],
      out_specs=pl.BlockSpec((128, 128), _const),
      out_shape=jax.ShapeDtypeStruct((128, 128), f32),
      scratch_shapes=[pltpu.VMEM((128, DH), f32), pltpu.VMEM((128, 1), f32)],
  )(h2, batp, Wc1, bc1r, gcr, becr, wc2p, bc2p)

  return outp[:B, :2]


# speedup vs baseline: 3.7032x; 1.0101x over previous
import functools

import jax
import jax.numpy as jnp
from jax import lax
from jax.experimental import pallas as pl
from jax.experimental.pallas import tpu as pltpu
from jax.experimental.pallas import tpu_sc as plsc

N = 10000
E = 320000
DIN = 128
DH = 256
B = 100

NP = 10240
EP = 327680
NC = 2
NS = 16
CHUNK = 128
CPT = EP // NS // CHUNK
RPT = NP // NS
FW = 64
TCB = 1024
NTCB = NP // TCB


KB = 4
SEG = 4
SCH = CPT // SEG
NG = SCH // KB


def _sweep(c, tab0, tab1, srcb, dstb, rowsA, rowsB, semA, semB, acc,
           deg_pair=None):
  def fire(bank, sem, g):
    for b in range(KB):
      ch = g * KB + b

      @pl.when(c == 0)
      def _():
        pltpu.async_copy(tab0.at[srcb.at[ch]], bank.at[b], sem)

      @pl.when(c == 1)
      def _():
        pltpu.async_copy(tab1.at[srcb.at[ch]], bank.at[b], sem)

  def drain(bank, sem):
    for b in range(KB):
      pltpu.make_async_copy(tab0.at[srcb.at[0]], bank.at[b], sem).wait()

  def scatter(bank, g):
    for b in range(KB):
      ch = g * KB + b
      pltpu.sync_copy(bank.at[b], acc.at[dstb.at[ch]], add=True)
      if deg_pair is not None:
        ones, dacc = deg_pair

        @pl.when(c == 0)
        def _():
          pltpu.sync_copy(ones, dacc.at[dstb.at[ch]], add=True)

  fire(rowsA, semA, 0)

  def body(j2, carry):
    g0 = 2 * j2
    drain(rowsA, semA)
    fire(rowsB, semB, g0 + 1)
    scatter(rowsA, g0)
    drain(rowsB, semB)

    @pl.when(j2 + 1 < NG // 2)
    def _():
      fire(rowsA, semA, g0 + 2)

    scatter(rowsB, g0 + 1)
    return carry

  lax.fori_loop(0, NG // 2, body, 0)


def _sage1_body(xa_ref, xb_ref, src_ref, dst_ref, z64_ref, z8_ref, ones_ref,
                suma_ref, sumb_ref, deg_ref, srcb, dstb, rowsA, rowsB, ones,
                acc, dacc, semA, semB):
  c = lax.axis_index("c")
  s = lax.axis_index("s")
  rsl = pl.ds(s * RPT, RPT)
  pltpu.sync_copy(z64_ref.at[rsl], acc.at[rsl])
  pltpu.sync_copy(z8_ref.at[rsl], dacc.at[rsl])
  pltpu.sync_copy(ones_ref, ones)
  plsc.subcore_barrier()

  for seg in range(SEG):
    csl = pl.ds(s * CPT + seg * SCH, SCH)
    pltpu.sync_copy(src_ref.at[csl], srcb)
    pltpu.sync_copy(dst_ref.at[csl], dstb)
    _sweep(c, xa_ref, xb_ref, srcb, dstb, rowsA, rowsB, semA, semB, acc,
           deg_pair=(ones, dacc))
  plsc.subcore_barrier()

  @pl.when(c == 0)
  def _():
    pltpu.sync_copy(acc.at[rsl], suma_ref.at[rsl])
    pltpu.sync_copy(dacc.at[rsl], deg_ref.at[rsl])

  @pl.when(c == 1)
  def _():
    pltpu.sync_copy(acc.at[rsl], sumb_ref.at[rsl])


def _sage2_body(h0_ref, h1_ref, h2_ref, h3_ref, src_ref, dst_ref, z64_ref,
                s2a_ref, s2b_ref, s2c_ref, s2d_ref, srcb, dstb, rowsA, rowsB,
                acc, semA, semB):
  c = lax.axis_index("c")
  s = lax.axis_index("s")
  rsl = pl.ds(s * RPT, RPT)

  outs = (s2a_ref, s2b_ref, s2c_ref, s2d_ref)
  for p in range(2):
    tab_c0 = h0_ref if p == 0 else h1_ref
    tab_c1 = h2_ref if p == 0 else h3_ref
    pltpu.sync_copy(z64_ref.at[rsl], acc.at[rsl])
    plsc.subcore_barrier()
    for seg in range(SEG):
      csl = pl.ds(s * CPT + seg * SCH, SCH)
      pltpu.sync_copy(src_ref.at[csl], srcb)
      pltpu.sync_copy(dst_ref.at[csl], dstb)
      _sweep(c, tab_c0, tab_c1, srcb, dstb, rowsA, rowsB, semA, semB, acc)
    plsc.subcore_barrier()

    @pl.when(c == 0)
    def _():
      pltpu.sync_copy(acc.at[rsl], outs[p].at[rsl])

    @pl.when(c == 1)
    def _():
      pltpu.sync_copy(acc.at[rsl], outs[2 + p].at[rsl])


@functools.cache
def _get_sc_kernels():
  mesh = plsc.VectorSubcoreMesh(
      core_axis_name="c", subcore_axis_name="s",
      num_cores=NC, num_subcores=NS)
  sage1 = pl.kernel(
      _sage1_body,
      out_type=[
          jax.ShapeDtypeStruct((NP, FW), jnp.float32),
          jax.ShapeDtypeStruct((NP, FW), jnp.float32),
          jax.ShapeDtypeStruct((NP, 8), jnp.float32),
      ],
      mesh=mesh,
      compiler_params=pltpu.CompilerParams(use_tc_tiling_on_sc=False),
      scratch_types=[
          pltpu.VMEM((SCH, CHUNK), jnp.int32),
          pltpu.VMEM((SCH, CHUNK), jnp.int32),
          pltpu.VMEM((KB, CHUNK, FW), jnp.float32),
          pltpu.VMEM((KB, CHUNK, FW), jnp.float32),
          pltpu.VMEM((CHUNK, 8), jnp.float32),
          pltpu.VMEM_SHARED((NP, FW), jnp.float32),
          pltpu.VMEM_SHARED((NP, 8), jnp.float32),
          pltpu.SemaphoreType.DMA,
          pltpu.SemaphoreType.DMA,
      ],
  )
  sage2 = pl.kernel(
      _sage2_body,
      out_type=[jax.ShapeDtypeStruct((NP, FW), jnp.float32)] * 4,
      mesh=mesh,
      compiler_params=pltpu.CompilerParams(use_tc_tiling_on_sc=False),
      scratch_types=[
          pltpu.VMEM((SCH, CHUNK), jnp.int32),
          pltpu.VMEM((SCH, CHUNK), jnp.int32),
          pltpu.VMEM((KB, CHUNK, FW), jnp.float32),
          pltpu.VMEM((KB, CHUNK, FW), jnp.float32),
          pltpu.VMEM_SHARED((NP, FW), jnp.float32),
          pltpu.SemaphoreType.DMA,
          pltpu.SemaphoreType.DMA,
      ],
  )
  return sage1, sage2


def _dbf(a, b):
  return jnp.dot(a.astype(jnp.bfloat16), b.astype(jnp.bfloat16),
                 preferred_element_type=jnp.float32)


def _row_mask(i, z):
  rid = i * TCB + lax.broadcasted_iota(jnp.int32, (TCB, 1), 0)
  return jnp.where(rid < N, z, 0.0)


def _acc_stats(i, st_ref, z):
  zm = _row_mask(i, z)
  upd = jnp.concatenate(
      [jnp.sum(zm, axis=0)[None, :], jnp.sum(zm * zm, axis=0)[None, :]], axis=0)

  @pl.when(i == 0)
  def _():
    st_ref[...] = upd

  @pl.when(i > 0)
  def _():
    st_ref[...] = st_ref[...] + upd


def _z1_body(sa_ref, sb_ref, d_ref, x_ref, wl_ref, wr_ref, b_ref,
             z_ref, st_ref):
  i = pl.program_id(0)
  rdeg = 1.0 / jnp.maximum(d_ref[:, 0:1], 1.0)
  z = (_dbf(sa_ref[...] * rdeg, wl_ref[0:FW, :])
       + _dbf(sb_ref[...] * rdeg, wl_ref[FW:, :])
       + _dbf(x_ref[...], wr_ref[...])
       + b_ref[...])
  z_ref[...] = z
  _acc_stats(i, st_ref, z)


def _bn_split_body(z_ref, st_ref, g_ref, be_ref,
                   h0_ref, h1_ref, h2_ref, h3_ref):
  m = st_ref[0:1, :] / float(N)
  var = st_ref[1:2, :] / float(N) - m * m
  rstd = (1.0 / jnp.sqrt(var + 1e-5))
  h = jnp.maximum((z_ref[...] - m) * rstd * g_ref[...] + be_ref[...], 0.0)
  h0_ref[...] = h[:, 0:FW]
  h1_ref[...] = h[:, FW:2 * FW]
  h2_ref[...] = h[:, 2 * FW:3 * FW]
  h3_ref[...] = h[:, 3 * FW:]


def _z2_body(sa_ref, sb_ref, sc_ref, sd_ref, d_ref,
             h0_ref, h1_ref, h2_ref, h3_ref,
             wl_ref, wr_ref, b_ref, z_ref, st_ref):
  i = pl.program_id(0)
  rdeg = 1.0 / jnp.maximum(d_ref[:, 0:1], 1.0)
  z = b_ref[...] + jnp.zeros((TCB, DH), jnp.float32)
  for g, (s_ref, h_ref) in enumerate(
      zip((sa_ref, sb_ref, sc_ref, sd_ref), (h0_ref, h1_ref, h2_ref, h3_ref))):
    wsl = slice(g * FW, (g + 1) * FW)
    z = z + _dbf(s_ref[...] * rdeg, wl_ref[wsl, :])
    z = z + _dbf(h_ref[...], wr_ref[wsl, :])
  z_ref[...] = z
  _acc_stats(i, st_ref, z)


def _bn_full_body(z_ref, st_ref, g_ref, be_ref, h_ref):
  m = st_ref[0:1, :] / float(N)
  var = st_ref[1:2, :] / float(N) - m * m
  rstd = (1.0 / jnp.sqrt(var + 1e-5))
  h_ref[...] = jnp.maximum(
      (z_ref[...] - m) * rstd * g_ref[...] + be_ref[...], 0.0)


def _head_body(h_ref, bat_ref, wc1_ref, bc1_ref, gc_ref, bec_ref,
               wc2_ref, bc2_ref, out_ref, pacc, cacc):
  i = pl.program_id(0)
  cols = lax.broadcasted_iota(jnp.int32, (TCB, 128), 1)
  p = (bat_ref[...] == cols).astype(jnp.float32)
  contrib = lax.dot_general(p, h_ref[...], (((0,), (0,)), ((), ())),
                            preferred_element_type=jnp.float32, precision=lax.Precision.HIGHEST)
  cnt = jnp.sum(p, axis=0)[:, None]

  @pl.when(i == 0)
  def _():
    pacc[...] = contrib
    cacc[...] = cnt

  @pl.when(i > 0)
  def _():
    pacc[...] = pacc[...] + contrib
    cacc[...] = cacc[...] + cnt

  @pl.when(i == NTCB - 1)
  def _():
    pooled = pacc[...] / jnp.maximum(cacc[...], 1.0)
    c1 = jnp.maximum(
        _dbf(pooled, wc1_ref[...])
        + bc1_ref[...], 0.0)
    rmask = lax.broadcasted_iota(jnp.int32, (128, 1), 0) < B
    c1 = jnp.where(rmask, c1, 0.0)
    m = jnp.sum(c1, axis=0, keepdims=True) / float(B)
    var = jnp.sum(c1 * c1, axis=0, keepdims=True) / float(B) - m * m
    cn = (c1 - m) * (1.0 / jnp.sqrt(var + 1e-5)) * gc_ref[...] + bec_ref[...]
    out_ref[...] = _dbf(cn, wc2_ref[...]) + bc2_ref[...]


def _rows(i):
  return (i, 0)


def _const(i):
  return (0, 0)


def kernel(x, edge_index, batch, W_l1, b1, W_r1, g1, be1, W_l2, b2, W_r2,
           g2, be2, Wc1, bc1, gc, bec, Wc2, bc2):
  f32 = jnp.float32
  src = jnp.concatenate(
      [edge_index[0].astype(jnp.int32), jnp.zeros((EP - E,), jnp.int32)])
  dst = jnp.concatenate(
      [edge_index[1].astype(jnp.int32),
       jnp.full((EP - E,), NP - 1, jnp.int32)])
  src2 = src.reshape(-1, CHUNK)
  dst2 = dst.reshape(-1, CHUNK)
  xp = jnp.concatenate([x, jnp.zeros((NP - N, DIN), f32)], axis=0)
  xa = xp[:, :FW]
  xb = xp[:, FW:]
  batp = jnp.concatenate(
      [batch.astype(jnp.int32), jnp.full((NP - N,), 1000, jnp.int32)]
  ).reshape(NP, 1)
  z64 = jnp.zeros((NP, FW), f32)
  z8 = jnp.zeros((NP, 8), f32)
  ones8 = jnp.ones((CHUNK, 8), f32)
  b1r = b1.reshape(1, DH)
  b2r = b2.reshape(1, DH)
  g1r, be1r = g1.reshape(1, DH), be1.reshape(1, DH)
  g2r, be2r = g2.reshape(1, DH), be2.reshape(1, DH)
  bc1r = bc1.reshape(1, DH)
  gcr, becr = gc.reshape(1, DH), bec.reshape(1, DH)
  wc2p = jnp.concatenate([Wc2, jnp.zeros((DH, 128 - Wc2.shape[1]), f32)], 1)
  bc2p = jnp.concatenate([bc2, jnp.zeros((128 - bc2.shape[0],), f32)]
                         ).reshape(1, 128)

  sage1, sage2 = _get_sc_kernels()
  s1a, s1b, deg = sage1(xa, xb, src2, dst2, z64, z8, ones8)

  hblk = pl.BlockSpec((TCB, DIN), _rows)
  gblk = pl.BlockSpec((TCB, FW), _rows)
  dblk = pl.BlockSpec((TCB, 8), _rows)
  zblk = pl.BlockSpec((TCB, DH), _rows)
  stblk = pl.BlockSpec((2, DH), _const)
  w128 = pl.BlockSpec((DIN, DH), _const)
  w256 = pl.BlockSpec((DH, DH), _const)
  vec = pl.BlockSpec((1, DH), _const)

  z1, st1 = pl.pallas_call(
      _z1_body,
      grid=(NTCB,),
      in_specs=[gblk, gblk, dblk, hblk, w128, w128, vec],
      out_specs=[zblk, stblk],
      out_shape=[jax.ShapeDtypeStruct((NP, DH), f32),
                 jax.ShapeDtypeStruct((2, DH), f32)],
  )(s1a, s1b, deg, xp, W_l1, W_r1, b1r)

  h1s = pl.pallas_call(
      _bn_split_body,
      grid=(NTCB,),
      in_specs=[zblk, stblk, vec, vec],
      out_specs=[gblk, gblk, gblk, gblk],
      out_shape=[jax.ShapeDtypeStruct((NP, FW), f32)] * 4,
  )(z1, st1, g1r, be1r)

  s2 = sage2(h1s[0], h1s[1], h1s[2], h1s[3], src2, dst2, z64)

  z2, st2 = pl.pallas_call(
      _z2_body,
      grid=(NTCB,),
      in_specs=[gblk, gblk, gblk, gblk, dblk,
                gblk, gblk, gblk, gblk, w256, w256, vec],
      out_specs=[zblk, stblk],
      out_shape=[jax.ShapeDtypeStruct((NP, DH), f32),
                 jax.ShapeDtypeStruct((2, DH), f32)],
  )(s2[0], s2[1], s2[2], s2[3], deg,
    h1s[0], h1s[1], h1s[2], h1s[3], W_l2, W_r2, b2r)

  h2 = pl.pallas_call(
      _bn_full_body,
      grid=(NTCB,),
      in_specs=[zblk, stblk, vec, vec],
      out_specs=zblk,
      out_shape=jax.ShapeDtypeStruct((NP, DH), f32),
  )(z2, st2, g2r, be2r)

  outp = pl.pallas_call(
      _head_body,
      grid=(NTCB,),
      in_specs=[zblk, pl.BlockSpec((TCB, 1), _rows), w256, vec, vec, vec,
                pl.BlockSpec((DH, 128), _const), pl.BlockSpec((1, 128), _const)],
      out_specs=pl.BlockSpec((128, 128), _const),
      out_shape=jax.ShapeDtypeStruct((128, 128), f32),
      scratch_shapes=[pltpu.VMEM((128, DH), f32), pltpu.VMEM((128, 1), f32)],
  )(h2, batp, Wc1, bc1r, gcr, becr, wc2p, bc2p)

  return outp[:B, :2]
